# baseline (speedup 1.0000x reference)
import functools

import jax
import jax.numpy as jnp
from jax import lax
from jax.experimental import pallas as pl
from jax.experimental.pallas import tpu as pltpu
from jax.experimental.pallas import tpu_sc as plsc

_BATCH = 1024
_TOKENS = 20
_HIDDEN = 2048
_N = _BATCH * _TOKENS

_NC = 2
_NS = 16
_NW = _NC * _NS
_SPLITS = 4
_NSPLIT = _N // _SPLITS
_ROWS_PER_W = _NSPLIT // _NW
_CHUNK = 16
_NCHUNK = _ROWS_PER_W // _CHUNK


def _gather_rows(weight, idx_flat):
    mesh = plsc.VectorSubcoreMesh(
        core_axis_name="core", subcore_axis_name="subcore"
    )

    @functools.partial(
        pl.kernel,
        out_type=jax.ShapeDtypeStruct((_NSPLIT, _HIDDEN), weight.dtype),
        mesh=mesh,
        scratch_types=[
            pltpu.VMEM((_ROWS_PER_W,), jnp.int32),
            pltpu.VMEM((_CHUNK, _HIDDEN), jnp.float32),
            pltpu.VMEM((_CHUNK, _HIDDEN), jnp.float32),
            pltpu.SemaphoreType.DMA,
            pltpu.SemaphoreType.DMA,
            pltpu.SemaphoreType.DMA,
            pltpu.SemaphoreType.DMA,
        ],
    )
    def gather_kernel(
        w_hbm, i_hbm, o_hbm, idx_v, buf0, buf1, gsem0, gsem1, wsem0, wsem1
    ):
        wid = lax.axis_index("subcore") * _NC + lax.axis_index("core")
        base = wid * _ROWS_PER_W
        pltpu.sync_copy(i_hbm.at[pl.ds(base, _ROWS_PER_W)], idx_v)

        def start_gather(c, buf, sem):
            pltpu.async_copy(w_hbm.at[idx_v.at[pl.ds(c * _CHUNK, _CHUNK)]], buf, sem)

        def start_write(c, buf, sem):
            pltpu.async_copy(buf, o_hbm.at[pl.ds(base + c * _CHUNK, _CHUNK)], sem)

        def wait_gather(buf, sem):
            pltpu.make_async_copy(w_hbm.at[pl.ds(0, _CHUNK)], buf, sem).wait()

        def wait_write(buf, sem):
            pltpu.make_async_copy(buf, o_hbm.at[pl.ds(base, _CHUNK)], sem).wait()

        start_gather(0, buf0, gsem0)
        start_gather(1, buf1, gsem1)

        @pl.loop(0, _NCHUNK, step=2)
        def _(c):
            wait_gather(buf0, gsem0)
            start_write(c, buf0, wsem0)
            wait_gather(buf1, gsem1)
            start_write(c + 1, buf1, wsem1)

            @pl.when(c + 2 < _NCHUNK)
            def _():
                wait_write(buf0, wsem0)
                start_gather(c + 2, buf0, gsem0)
                wait_write(buf1, wsem1)
                start_gather(c + 3, buf1, gsem1)

        wait_write(buf0, wsem0)
        wait_write(buf1, wsem1)

    return gather_kernel(weight, idx_flat)


def kernel(indices, weight):
    idx_flat = indices.reshape(_N)
    parts = [
        _gather_rows(weight, idx_flat[s * _NSPLIT:(s + 1) * _NSPLIT]).reshape(
            _BATCH // _SPLITS, _TOKENS, _HIDDEN
        )
        for s in range(_SPLITS)
    ]
    return jnp.concatenate(parts, axis=0)


# speedup vs baseline: 1.2081x; 1.2081x over previous
import functools

import jax
import jax.numpy as jnp
from jax import lax
from jax.experimental import pallas as pl
from jax.experimental.pallas import tpu as pltpu
from jax.experimental.pallas import tpu_sc as plsc

_BATCH = 1024
_TOKENS = 20
_HIDDEN = 2048
_N = _BATCH * _TOKENS

_NC = 2
_NS = 16
_NW = _NC * _NS
_ROWS_PER_W = _N // _NW
_CHUNK = 16
_NCHUNK = _ROWS_PER_W // _CHUNK

_BB = 32


def _gather_rows(weight, idx_flat):
    mesh = plsc.VectorSubcoreMesh(
        core_axis_name="core", subcore_axis_name="subcore"
    )

    @functools.partial(
        pl.kernel,
        out_type=jax.ShapeDtypeStruct((_N, _HIDDEN), weight.dtype),
        mesh=mesh,
        scratch_types=[
            pltpu.VMEM((_ROWS_PER_W,), jnp.int32),
            pltpu.VMEM((_CHUNK, _HIDDEN), jnp.float32),
            pltpu.VMEM((_CHUNK, _HIDDEN), jnp.float32),
            pltpu.SemaphoreType.DMA,
            pltpu.SemaphoreType.DMA,
            pltpu.SemaphoreType.DMA,
            pltpu.SemaphoreType.DMA,
        ],
    )
    def gather_kernel(
        w_hbm, i_hbm, o_hbm, idx_v, buf0, buf1, gsem0, gsem1, wsem0, wsem1
    ):
        wid = lax.axis_index("subcore") * _NC + lax.axis_index("core")
        base = wid * _ROWS_PER_W
        pltpu.sync_copy(i_hbm.at[pl.ds(base, _ROWS_PER_W)], idx_v)

        def start_gather(c, buf, sem):
            pltpu.async_copy(w_hbm.at[idx_v.at[pl.ds(c * _CHUNK, _CHUNK)]], buf, sem)

        def start_write(c, buf, sem):
            pltpu.async_copy(buf, o_hbm.at[pl.ds(base + c * _CHUNK, _CHUNK)], sem)

        def wait_gather(buf, sem):
            pltpu.make_async_copy(w_hbm.at[pl.ds(0, _CHUNK)], buf, sem).wait()

        def wait_write(buf, sem):
            pltpu.make_async_copy(buf, o_hbm.at[pl.ds(base, _CHUNK)], sem).wait()

        start_gather(0, buf0, gsem0)
        start_gather(1, buf1, gsem1)

        @pl.loop(0, _NCHUNK, step=2)
        def _(c):
            wait_gather(buf0, gsem0)
            start_write(c, buf0, wsem0)
            wait_gather(buf1, gsem1)
            start_write(c + 1, buf1, wsem1)

            @pl.when(c + 2 < _NCHUNK)
            def _():
                wait_write(buf0, wsem0)
                start_gather(c + 2, buf0, gsem0)
                wait_write(buf1, wsem1)
                start_gather(c + 3, buf1, gsem1)

        wait_write(buf0, wsem0)
        wait_write(buf1, wsem1)

    return gather_kernel(weight, idx_flat)


def _relayout_body(i_ref, o_ref):
    o_ref[...] = i_ref[...].reshape(_BB, _TOKENS, _HIDDEN)


def _relayout(flat):
    return pl.pallas_call(
        _relayout_body,
        grid=(_BATCH // _BB,),
        in_specs=[
            pl.BlockSpec((_BB * _TOKENS, _HIDDEN), lambda b: (b, 0)),
        ],
        out_specs=pl.BlockSpec(
            (_BB, _TOKENS, _HIDDEN), lambda b: (b, 0, 0)
        ),
        out_shape=jax.ShapeDtypeStruct(
            (_BATCH, _TOKENS, _HIDDEN), flat.dtype
        ),
    )(flat)


def kernel(indices, weight):
    idx_flat = indices.reshape(_N)
    flat = _gather_rows(weight, idx_flat)
    return _relayout(flat)


# speedup vs baseline: 2.2489x; 1.8616x over previous
import functools

import jax
import jax.numpy as jnp
from jax import lax
from jax.experimental import pallas as pl
from jax.experimental.pallas import tpu as pltpu
from jax.experimental.pallas import tpu_sc as plsc

_BATCH = 1024
_TOKENS = 20
_HIDDEN = 2048
_N = _BATCH * _TOKENS

_NC = 2
_NS = 16
_NW = _NC * _NS
_ROWS_PER_W = _N // _NW
_CHUNK = 16
_NCHUNK = _ROWS_PER_W // _CHUNK

_BB = 32


def _gather_rows(weight, idx_flat):
    mesh = plsc.VectorSubcoreMesh(
        core_axis_name="core", subcore_axis_name="subcore"
    )

    @functools.partial(
        pl.kernel,
        out_type=jax.ShapeDtypeStruct((_N, _HIDDEN), weight.dtype),
        mesh=mesh,
        compiler_params=pltpu.CompilerParams(use_tc_tiling_on_sc=True),
        scratch_types=[
            pltpu.VMEM((_ROWS_PER_W,), jnp.int32),
            pltpu.VMEM((_CHUNK, _HIDDEN), jnp.float32),
            pltpu.VMEM((_CHUNK, _HIDDEN), jnp.float32),
            pltpu.SemaphoreType.DMA,
            pltpu.SemaphoreType.DMA,
            pltpu.SemaphoreType.DMA,
            pltpu.SemaphoreType.DMA,
        ],
    )
    def gather_kernel(
        w_hbm, i_hbm, o_hbm, idx_v, buf0, buf1, gsem0, gsem1, wsem0, wsem1
    ):
        wid = lax.axis_index("subcore") * _NC + lax.axis_index("core")
        base = wid * _ROWS_PER_W
        pltpu.sync_copy(i_hbm.at[pl.ds(base, _ROWS_PER_W)], idx_v)

        def start_gather(c, buf, sem):
            pltpu.async_copy(w_hbm.at[idx_v.at[pl.ds(c * _CHUNK, _CHUNK)]], buf, sem)

        def start_write(c, buf, sem):
            pltpu.async_copy(buf, o_hbm.at[pl.ds(base + c * _CHUNK, _CHUNK)], sem)

        def wait_gather(buf, sem):
            pltpu.make_async_copy(w_hbm.at[pl.ds(0, _CHUNK)], buf, sem).wait()

        def wait_write(buf, sem):
            pltpu.make_async_copy(buf, o_hbm.at[pl.ds(base, _CHUNK)], sem).wait()

        start_gather(0, buf0, gsem0)
        start_gather(1, buf1, gsem1)

        @pl.loop(0, _NCHUNK, step=2)
        def _(c):
            wait_gather(buf0, gsem0)
            start_write(c, buf0, wsem0)
            wait_gather(buf1, gsem1)
            start_write(c + 1, buf1, wsem1)

            @pl.when(c + 2 < _NCHUNK)
            def _():
                wait_write(buf0, wsem0)
                start_gather(c + 2, buf0, gsem0)
                wait_write(buf1, wsem1)
                start_gather(c + 3, buf1, gsem1)

        wait_write(buf0, wsem0)
        wait_write(buf1, wsem1)

    return gather_kernel(weight, idx_flat)


def _relayout_body(i_ref, o_ref):
    o_ref[...] = i_ref[...].reshape(_BB, _TOKENS, _HIDDEN)


def _relayout(flat):
    return pl.pallas_call(
        _relayout_body,
        grid=(_BATCH // _BB,),
        in_specs=[
            pl.BlockSpec((_BB * _TOKENS, _HIDDEN), lambda b: (b, 0)),
        ],
        out_specs=pl.BlockSpec(
            (_BB, _TOKENS, _HIDDEN), lambda b: (b, 0, 0)
        ),
        out_shape=jax.ShapeDtypeStruct(
            (_BATCH, _TOKENS, _HIDDEN), flat.dtype
        ),
    )(flat)


def kernel(indices, weight):
    idx_flat = indices.T.reshape(_N)
    flat = _gather_rows(weight, idx_flat)
    return flat.reshape(_TOKENS, _BATCH, _HIDDEN).transpose(1, 0, 2)


# speedup vs baseline: 2.2505x; 1.0007x over previous
import functools

import jax
import jax.numpy as jnp
from jax import lax
from jax.experimental import pallas as pl
from jax.experimental.pallas import tpu as pltpu
from jax.experimental.pallas import tpu_sc as plsc

_BATCH = 1024
_TOKENS = 20
_HIDDEN = 2048
_N = _BATCH * _TOKENS

_NC = 2
_NS = 16
_NW = _NC * _NS
_ROWS_PER_W = _N // _NW
_CHUNK = 40
_NCHUNK = _ROWS_PER_W // _CHUNK

_BB = 32


def _gather_rows(weight, idx_flat):
    mesh = plsc.VectorSubcoreMesh(
        core_axis_name="core", subcore_axis_name="subcore"
    )

    @functools.partial(
        pl.kernel,
        out_type=jax.ShapeDtypeStruct((_N, _HIDDEN), weight.dtype),
        mesh=mesh,
        compiler_params=pltpu.CompilerParams(use_tc_tiling_on_sc=True),
        scratch_types=[
            pltpu.VMEM((_ROWS_PER_W,), jnp.int32),
            pltpu.VMEM((_CHUNK, _HIDDEN), jnp.float32),
            pltpu.SemaphoreType.DMA,
        ],
    )
    def gather_kernel(w_hbm, i_hbm, o_hbm, idx_v, buf0, gsem0):
        wid = lax.axis_index("subcore") * _NC + lax.axis_index("core")
        base = wid * _ROWS_PER_W
        pltpu.sync_copy(i_hbm.at[pl.ds(base, _ROWS_PER_W)], idx_v)

        @pl.loop(0, _NCHUNK)
        def _(c):
            pltpu.async_copy(
                w_hbm.at[idx_v.at[pl.ds(c * _CHUNK, _CHUNK)]], buf0, gsem0
            ).wait()
            pltpu.sync_copy(buf0, o_hbm.at[pl.ds(base + c * _CHUNK, _CHUNK)])

    return gather_kernel(weight, idx_flat)


def _relayout_body(i_ref, o_ref):
    o_ref[...] = i_ref[...].reshape(_BB, _TOKENS, _HIDDEN)


def _relayout(flat):
    return pl.pallas_call(
        _relayout_body,
        grid=(_BATCH // _BB,),
        in_specs=[
            pl.BlockSpec((_BB * _TOKENS, _HIDDEN), lambda b: (b, 0)),
        ],
        out_specs=pl.BlockSpec(
            (_BB, _TOKENS, _HIDDEN), lambda b: (b, 0, 0)
        ),
        out_shape=jax.ShapeDtypeStruct(
            (_BATCH, _TOKENS, _HIDDEN), flat.dtype
        ),
    )(flat)


def kernel(indices, weight):
    idx_flat = indices.T.reshape(_N)
    flat = _gather_rows(weight, idx_flat)
    return flat.reshape(_TOKENS, _BATCH, _HIDDEN).transpose(1, 0, 2)


# speedup vs baseline: 2.2560x; 1.0025x over previous
import functools

import jax
import jax.numpy as jnp
from jax import lax
from jax.experimental import pallas as pl
from jax.experimental.pallas import tpu as pltpu
from jax.experimental.pallas import tpu_sc as plsc

_BATCH = 1024
_TOKENS = 20
_HIDDEN = 2048
_N = _BATCH * _TOKENS

_NC = 2
_NS = 16
_NW = _NC * _NS
_ROWS_PER_W = _N // _NW
_CHUNK = 16
_NCHUNK = _ROWS_PER_W // _CHUNK


def _gather_rows(weight, idx_flat):
    mesh = plsc.VectorSubcoreMesh(
        core_axis_name="core", subcore_axis_name="subcore"
    )

    @functools.partial(
        pl.kernel,
        out_type=jax.ShapeDtypeStruct((_N, _HIDDEN), weight.dtype),
        mesh=mesh,
        compiler_params=pltpu.CompilerParams(use_tc_tiling_on_sc=True),
        scratch_types=[
            pltpu.VMEM((_ROWS_PER_W,), jnp.int32),
            pltpu.VMEM((_CHUNK, _HIDDEN), jnp.float32),
            pltpu.VMEM((_CHUNK, _HIDDEN), jnp.float32),
            pltpu.SemaphoreType.DMA,
            pltpu.SemaphoreType.DMA,
            pltpu.SemaphoreType.DMA,
            pltpu.SemaphoreType.DMA,
        ],
    )
    def gather_kernel(
        w_hbm, i_hbm, o_hbm, idx_v, buf0, buf1, gsem0, gsem1, wsem0, wsem1
    ):
        wid = lax.axis_index("subcore") * _NC + lax.axis_index("core")
        base = wid * _ROWS_PER_W
        pltpu.sync_copy(i_hbm.at[pl.ds(base, _ROWS_PER_W)], idx_v)

        def start_gather(c, buf, sem):
            pltpu.async_copy(
                w_hbm.at[idx_v.at[pl.ds(c * _CHUNK, _CHUNK)]], buf, sem
            )

        def start_write(c, buf, sem):
            pltpu.async_copy(buf, o_hbm.at[pl.ds(base + c * _CHUNK, _CHUNK)], sem)

        def wait_gather(buf, sem):
            pltpu.make_async_copy(w_hbm.at[pl.ds(0, _CHUNK)], buf, sem).wait()

        def wait_write(buf, sem):
            pltpu.make_async_copy(buf, o_hbm.at[pl.ds(base, _CHUNK)], sem).wait()

        start_gather(0, buf0, gsem0)
        start_gather(1, buf1, gsem1)

        @pl.loop(0, _NCHUNK, step=2)
        def _(c):
            wait_gather(buf0, gsem0)
            start_write(c, buf0, wsem0)
            wait_gather(buf1, gsem1)
            start_write(c + 1, buf1, wsem1)

            @pl.when(c + 2 < _NCHUNK)
            def _():
                wait_write(buf0, wsem0)
                start_gather(c + 2, buf0, gsem0)
                wait_write(buf1, wsem1)
                start_gather(c + 3, buf1, gsem1)

        wait_write(buf0, wsem0)
        wait_write(buf1, wsem1)

    return gather_kernel(weight, idx_flat)


def kernel(indices, weight):
    idx_flat = indices.T.reshape(_N)
    flat = _gather_rows(weight, idx_flat)
    return flat.reshape(_TOKENS, _BATCH, _HIDDEN).transpose(1, 0, 2)
